# baseline (speedup 1.0000x reference)
import jax
import jax.numpy as jnp
from jax import lax
from jax.experimental import pallas as pl
from jax.experimental.pallas import tpu as pltpu
from jax.experimental.pallas import tpu_sc as plsc

MIN_BOX_F = 0.05
NC, NS, L = 2, 16, 16
B, H, W = 32, 512, 512
NJ = W // L
R = 32
NCHUNK = H // R
BIGF = float(H + W)


def _tec_body(mask_hbm, out_hbm, buf0, buf1, outv, sem0, sem1):
    wid = lax.axis_index("s") * NC + lax.axis_index("c")
    bufs = (buf0, buf1)
    sems = (sem0, sem1)

    ninf = jnp.full((L,), -jnp.inf, jnp.float32)
    bigv = jnp.full((L,), BIGF, jnp.float32)
    negv = jnp.full((L,), -1.0, jnp.float32)
    colacc = tuple(ninf for _ in range(NJ))

    def make_row_body(buf, base):
        def row_body(r, carry):
            colacc, yminv, ymaxv = carry
            new = []
            rm = [None, None, None, None]
            for j in range(NJ):
                v = buf[r, pl.ds(j * L, L)]
                new.append(jnp.maximum(colacc[j], v))
                k = j % 4
                rm[k] = v if j < 4 else jnp.maximum(rm[k], v)
            rowv = jnp.maximum(jnp.maximum(rm[0], rm[1]),
                               jnp.maximum(rm[2], rm[3]))
            m = rowv > 0.5
            rf = (r + base).astype(jnp.float32)
            yminv = jnp.minimum(yminv, jnp.where(m, rf, BIGF))
            ymaxv = jnp.maximum(ymaxv, jnp.where(m, rf, -1.0))
            return tuple(new), yminv, ymaxv
        return row_body

    pend = [None, None]
    pend[0] = pltpu.async_copy(mask_hbm.at[wid, pl.ds(0, R)], buf0, sem0)
    carry = (colacc, bigv, negv)
    for c in range(NCHUNK):
        nxt = c + 1
        if nxt < NCHUNK:
            pend[nxt % 2] = pltpu.async_copy(
                mask_hbm.at[wid, pl.ds(nxt * R, R)], bufs[nxt % 2],
                sems[nxt % 2])
        pend[c % 2].wait()
        carry = lax.fori_loop(0, R, make_row_body(bufs[c % 2], c * R), carry)
    colacc, yminv, ymaxv = carry

    lanef = lax.iota(jnp.int32, L).astype(jnp.float32)
    xminv, xmaxv = bigv, negv
    for j in range(NJ):
        m = colacc[j] > 0.5
        idx = lanef + float(j * L)
        xminv = jnp.minimum(xminv, jnp.where(m, idx, BIGF))
        xmaxv = jnp.maximum(xmaxv, jnp.where(m, idx, -1.0))

    x_min = lax.full((L,), jnp.min(xminv), jnp.float32) * (1.0 / W)
    x_max = lax.full((L,), jnp.max(xmaxv), jnp.float32) * (1.0 / W)
    y_min = lax.full((L,), jnp.min(yminv), jnp.float32) * (1.0 / H)
    y_max = lax.full((L,), jnp.max(ymaxv), jnp.float32) * (1.0 / H)

    has_fg = x_max >= 0.0

    cond_x = (x_max - x_min) < MIN_BOX_F
    x_center = (x_min + x_max) * 0.5
    x_min = jnp.where(cond_x, jnp.maximum(0.0, x_center - MIN_BOX_F / 2),
                      x_min)
    x_max = jnp.where(cond_x, jnp.minimum(1.0, x_center + MIN_BOX_F / 2),
                      x_max)
    cond_y = (y_max - y_min) < MIN_BOX_F
    y_center = (y_min + y_max) * 0.5
    y_min = jnp.where(cond_y, jnp.maximum(0.0, y_center - MIN_BOX_F / 2),
                      y_min)
    y_max = jnp.where(cond_y, jnp.minimum(1.0, y_center + MIN_BOX_F / 2),
                      y_max)

    lane = lax.iota(jnp.int32, L)
    res = jnp.where(lane == 0, x_min,
                    jnp.where(lane == 1, y_min,
                              jnp.where(lane == 2, x_max, y_max)))
    default = jnp.where(lane < 2, 0.25, 0.75)
    res = jnp.where(has_fg, res, default)

    outv[...] = res
    pltpu.sync_copy(outv, out_hbm.at[wid])


@jax.jit
def _bbox_sc(mask):
    mesh = plsc.VectorSubcoreMesh(core_axis_name="c", subcore_axis_name="s",
                                  num_cores=NC, num_subcores=NS)
    f = pl.kernel(
        _tec_body,
        out_type=jax.ShapeDtypeStruct((B, L), jnp.float32),
        mesh=mesh,
        compiler_params=pltpu.CompilerParams(needs_layout_passes=False),
        scratch_types=[
            pltpu.VMEM((R, W), jnp.float32),
            pltpu.VMEM((R, W), jnp.float32),
            pltpu.VMEM((L,), jnp.float32),
            pltpu.SemaphoreType.DMA,
            pltpu.SemaphoreType.DMA,
        ],
    )
    return f(mask)


def kernel(mask_fg):
    assert mask_fg.shape == (B, 1, H, W), mask_fg.shape
    out = _bbox_sc(mask_fg.reshape(B, H, W))
    return out[:, :4]
